# baseline (speedup 1.0000x reference)
import jax
import jax.numpy as jnp
from jax import lax
from jax.experimental import pallas as pl
from jax.experimental.pallas import tpu as pltpu
from jax.experimental.pallas import tpu_sc as plsc

_NUM_HISTOGRAM = 1000
_EMBED = 16
_LOWER = 0.0
_STEP = (1.0 - 0.0) / _NUM_HISTOGRAM

_B, _L = 16384, 200
_NC, _NS = 2, 16
_NW = _NC * _NS
_ROWS_W = _B // _NW
_RCHUNK = 16
_NCHUNK = _ROWS_W // _RCHUNK
_LANES = 16
_GOFFS = list(range(0, _L - _LANES + 1, _LANES)) + [_L - _LANES]


def _body(scores_hbm, table_hbm, out_hbm, s_v, table_v, rows_v, sem):
    pltpu.sync_copy(table_hbm, table_v)
    wid = lax.axis_index("s") * _NC + lax.axis_index("c")
    row0 = wid * _ROWS_W
    iota16 = lax.iota(jnp.int32, _LANES)

    def chunk_body(ci, carry):
        r0 = row0 + ci * _RCHUNK
        pltpu.sync_copy(scores_hbm.at[pl.ds(r0, _RCHUNK)], s_v)

        def row_body(ri, c):
            vrow = jnp.full((_LANES,), 0, jnp.int32) + ri
            for goff in _GOFFS:
                s = s_v[ri, pl.ds(goff, _LANES)]
                gidx = ((s - _LOWER) / _STEP).astype(jnp.int32) * _EMBED
                vl = iota16 + goff
                for col in range(_EMBED):
                    vals = plsc.load_gather(table_v, [gidx])
                    plsc.store_scatter(
                        rows_v,
                        [vrow, vl, jnp.full((_LANES,), col, jnp.int32)],
                        vals,
                    )
                    if col + 1 < _EMBED:
                        gidx = gidx + 1
            return c

        lax.fori_loop(0, _RCHUNK, row_body, 0)
        pltpu.sync_copy(rows_v, out_hbm.at[pl.ds(r0, _RCHUNK)])
        return carry

    lax.fori_loop(0, _NCHUNK, chunk_body, 0)


def kernel(scores, table):
    f = pl.kernel(
        _body,
        out_type=jax.ShapeDtypeStruct((_B, _L, _EMBED), jnp.float32),
        mesh=plsc.VectorSubcoreMesh(core_axis_name="c", subcore_axis_name="s"),
        compiler_params=pltpu.CompilerParams(
            needs_layout_passes=False, use_tc_tiling_on_sc=False
        ),
        scratch_types=[
            pltpu.VMEM((_RCHUNK, _L), jnp.float32),
            pltpu.VMEM((_NUM_HISTOGRAM * _EMBED,), jnp.float32),
            pltpu.VMEM((_RCHUNK, _L, _EMBED), jnp.float32),
            pltpu.SemaphoreType.DMA,
        ],
    )
    return f(scores, table.reshape(_NUM_HISTOGRAM * _EMBED))


# speedup vs baseline: 2.6429x; 2.6429x over previous
import jax
import jax.numpy as jnp
from jax import lax
from jax.experimental import pallas as pl
from jax.experimental.pallas import tpu as pltpu
from jax.experimental.pallas import tpu_sc as plsc

_NUM_HISTOGRAM = 1000
_EMBED = 16
_LOWER = 0.0
_STEP = (1.0 - 0.0) / _NUM_HISTOGRAM

_B, _L = 16384, 200
_NC, _NS = 2, 16
_NW = _NC * _NS
_BBLK = _B // _NW
_LCH = 8
_NCHUNK = _L // _LCH
_LANES = 16
_NG = _BBLK // _LANES


def _body(scores_hbm, table_hbm, out_hbm, s_v, table_v, rows_v, sem):
    pltpu.sync_copy(table_hbm, table_v)
    wid = lax.axis_index("s") * _NC + lax.axis_index("c")
    b0 = wid * _BBLK
    iota16 = lax.iota(jnp.int32, _LANES)

    def chunk_body(ci, carry):
        l0 = ci * _LCH
        pltpu.sync_copy(
            scores_hbm.at[pl.ds(l0, _LCH), pl.ds(b0, _BBLK)], s_v
        )

        def l_body(li, c):
            vl = jnp.full((_LANES,), 0, jnp.int32) + li
            for g in range(_NG):
                s = s_v[li, pl.ds(g * _LANES, _LANES)]
                gidx = ((s - _LOWER) / _STEP).astype(jnp.int32) * _EMBED
                vb = iota16 + g * _LANES
                for col in range(_EMBED):
                    vals = plsc.load_gather(table_v, [gidx])
                    plsc.store_scatter(
                        rows_v,
                        [vl, jnp.full((_LANES,), col, jnp.int32), vb],
                        vals,
                    )
                    if col + 1 < _EMBED:
                        gidx = gidx + 1
            return c

        lax.fori_loop(0, _LCH, l_body, 0)
        pltpu.sync_copy(
            rows_v, out_hbm.at[pl.ds(l0, _LCH), pl.ds(0, _EMBED), pl.ds(b0, _BBLK)]
        )
        return carry

    lax.fori_loop(0, _NCHUNK, chunk_body, 0)


def kernel(scores, table):
    f = pl.kernel(
        _body,
        out_type=jax.ShapeDtypeStruct((_L, _EMBED, _B), jnp.float32),
        mesh=plsc.VectorSubcoreMesh(core_axis_name="c", subcore_axis_name="s"),
        compiler_params=pltpu.CompilerParams(needs_layout_passes=False),
        scratch_types=[
            pltpu.VMEM((_LCH, _BBLK), jnp.float32),
            pltpu.VMEM((_NUM_HISTOGRAM * _EMBED,), jnp.float32),
            pltpu.VMEM((_LCH, _EMBED, _BBLK), jnp.float32),
            pltpu.SemaphoreType.DMA,
        ],
    )
    out_t = f(scores.T, table.reshape(_NUM_HISTOGRAM * _EMBED))
    return jnp.transpose(out_t, (2, 0, 1))


# speedup vs baseline: 5.7420x; 2.1726x over previous
import jax
import jax.numpy as jnp
from jax import lax
from jax.experimental import pallas as pl
from jax.experimental.pallas import tpu as pltpu
from jax.experimental.pallas import tpu_sc as plsc

_NUM_HISTOGRAM = 1000
_EMBED = 16
_LOWER = 0.0
_STEP = (1.0 - 0.0) / _NUM_HISTOGRAM

_B, _L = 16384, 200
_NC, _NS = 2, 16
_NW = _NC * _NS
_BBLK = _B // _NW
_LCH = 8
_NCHUNK = _L // _LCH
_LANES = 16
_NG = _BBLK // _LANES


def _body(scores_hbm, table_hbm, out_hbm, s_v, table_v, rows_v, sem):
    pltpu.sync_copy(table_hbm, table_v)
    wid = lax.axis_index("s") * _NC + lax.axis_index("c")
    b0 = wid * _BBLK
    iota16 = lax.iota(jnp.int32, _LANES)

    def chunk_body(ci, carry):
        l0 = ci * _LCH
        pltpu.sync_copy(
            scores_hbm.at[pl.ds(l0, _LCH), pl.ds(b0, _BBLK)], s_v
        )

        def l_body(li, c):
            for g in range(_NG):
                s = s_v[li, pl.ds(g * _LANES, _LANES)]
                base = ((s - _LOWER) / _STEP).astype(jnp.int32) * _EMBED
                vals = [
                    plsc.load_gather(table_v, [base + col])
                    for col in range(_EMBED)
                ]
                for col in range(_EMBED):
                    rows_v[li, col, pl.ds(g * _LANES, _LANES)] = vals[col]
            return c

        lax.fori_loop(0, _LCH, l_body, 0)
        pltpu.sync_copy(
            rows_v, out_hbm.at[pl.ds(l0, _LCH), pl.ds(0, _EMBED), pl.ds(b0, _BBLK)]
        )
        return carry

    lax.fori_loop(0, _NCHUNK, chunk_body, 0)


def kernel(scores, table):
    f = pl.kernel(
        _body,
        out_type=jax.ShapeDtypeStruct((_L, _EMBED, _B), jnp.float32),
        mesh=plsc.VectorSubcoreMesh(core_axis_name="c", subcore_axis_name="s"),
        compiler_params=pltpu.CompilerParams(needs_layout_passes=False),
        scratch_types=[
            pltpu.VMEM((_LCH, _BBLK), jnp.float32),
            pltpu.VMEM((_NUM_HISTOGRAM * _EMBED,), jnp.float32),
            pltpu.VMEM((_LCH, _EMBED, _BBLK), jnp.float32),
            pltpu.SemaphoreType.DMA,
        ],
    )
    out_t = f(scores.T, table.reshape(_NUM_HISTOGRAM * _EMBED))
    return jnp.transpose(out_t, (2, 0, 1))


# speedup vs baseline: 6.3714x; 1.1096x over previous
import jax
import jax.numpy as jnp
from jax import lax
from jax.experimental import pallas as pl
from jax.experimental.pallas import tpu as pltpu
from jax.experimental.pallas import tpu_sc as plsc

_NUM_HISTOGRAM = 1000
_EMBED = 16
_LOWER = 0.0
_STEP = (1.0 - 0.0) / _NUM_HISTOGRAM

_B, _L = 16384, 200
_NC, _NS = 2, 16
_NW = _NC * _NS
_BBLK = _B // _NW
_LCH = 4
_NCHUNK = _L // _LCH
_LANES = 16
_NG = _BBLK // _LANES


def _body(scores_hbm, table_hbm, out_hbm, s_bufs, rows_bufs, table_v, sem_s, sem_o):
    pltpu.sync_copy(table_hbm, table_v)
    wid = lax.axis_index("s") * _NC + lax.axis_index("c")
    b0 = wid * _BBLK

    def start_s(ci, buf):
        pltpu.async_copy(
            scores_hbm.at[pl.ds(ci * _LCH, _LCH), pl.ds(b0, _BBLK)], buf, sem_s
        )

    def wait_s(buf):
        pltpu.make_async_copy(
            scores_hbm.at[pl.ds(0, _LCH), pl.ds(b0, _BBLK)], buf, sem_s
        ).wait()

    def start_o(ci, buf):
        pltpu.async_copy(
            buf,
            out_hbm.at[pl.ds(ci * _LCH, _LCH), pl.ds(0, _EMBED), pl.ds(b0, _BBLK)],
            sem_o,
        )

    def wait_o(buf):
        pltpu.make_async_copy(
            buf,
            out_hbm.at[pl.ds(0, _LCH), pl.ds(0, _EMBED), pl.ds(b0, _BBLK)],
            sem_o,
        ).wait()

    def compute(s_v, rows_v):
        def l_body(li, c):
            for g in range(_NG):
                s = s_v[li, pl.ds(g * _LANES, _LANES)]
                base = ((s - _LOWER) / _STEP).astype(jnp.int32) * _EMBED
                vals = [
                    plsc.load_gather(table_v, [base + col])
                    for col in range(_EMBED)
                ]
                for col in range(_EMBED):
                    rows_v[li, col, pl.ds(g * _LANES, _LANES)] = vals[col]
            return c

        lax.fori_loop(0, _LCH, l_body, 0)

    start_s(0, s_bufs[0])
    start_s(1, s_bufs[1])
    for k in range(2):
        wait_s(s_bufs[k])
        compute(s_bufs[k], rows_bufs[k])
        start_o(k, rows_bufs[k])
        start_s(k + 2, s_bufs[k])

    def pair_body(pi, carry):
        for k in range(2):
            ci = pi * 2 + k
            wait_s(s_bufs[k])
            wait_o(rows_bufs[k])
            compute(s_bufs[k], rows_bufs[k])
            start_o(ci, rows_bufs[k])

            @pl.when(ci + 2 < _NCHUNK)
            def _():
                start_s(ci + 2, s_bufs[k])

        return carry

    lax.fori_loop(1, _NCHUNK // 2, pair_body, 0)
    wait_o(rows_bufs[0])
    wait_o(rows_bufs[1])


def kernel(scores, table):
    f = pl.kernel(
        _body,
        out_type=jax.ShapeDtypeStruct((_L, _EMBED, _B), jnp.float32),
        mesh=plsc.VectorSubcoreMesh(core_axis_name="c", subcore_axis_name="s"),
        compiler_params=pltpu.CompilerParams(needs_layout_passes=False),
        scratch_types=[
            [pltpu.VMEM((_LCH, _BBLK), jnp.float32) for _ in range(2)],
            [pltpu.VMEM((_LCH, _EMBED, _BBLK), jnp.float32) for _ in range(2)],
            pltpu.VMEM((_NUM_HISTOGRAM * _EMBED,), jnp.float32),
            pltpu.SemaphoreType.DMA,
            pltpu.SemaphoreType.DMA,
        ],
    )
    out_t = f(scores.T, table.reshape(_NUM_HISTOGRAM * _EMBED))
    return jnp.transpose(out_t, (2, 0, 1))


# speedup vs baseline: 9.3625x; 1.4695x over previous
import jax
import jax.numpy as jnp
from jax import lax
from jax.experimental import pallas as pl
from jax.experimental.pallas import tpu as pltpu
from jax.experimental.pallas import tpu_sc as plsc

_NUM_HISTOGRAM = 1000
_EMBED = 16
_LOWER = 0.0
_STEP = (1.0 - 0.0) / _NUM_HISTOGRAM

_B, _L = 16384, 200
_NC, _NS = 2, 16
_NW = _NC * _NS
_BBLK = _B // _NW
_LCH = 4
_NCHUNK = _L // _LCH
_LANES = 16
_NG = _BBLK // _LANES


def _body(scores_hbm, table_hbm, out_hbm, s_bufs, rows_bufs, table_v, sem_s, sem_o):
    pltpu.sync_copy(table_hbm, table_v)
    wid = lax.axis_index("s") * _NC + lax.axis_index("c")
    b0 = wid * _BBLK

    def start_s(ci, buf):
        pltpu.async_copy(
            scores_hbm.at[pl.ds(ci * _LCH, _LCH), pl.ds(b0, _BBLK)], buf, sem_s
        )

    def wait_s(buf):
        pltpu.make_async_copy(
            scores_hbm.at[pl.ds(0, _LCH), pl.ds(b0, _BBLK)], buf, sem_s
        ).wait()

    def start_o(ci, buf):
        pltpu.async_copy(
            buf,
            out_hbm.at[pl.ds(ci * _LCH, _LCH), pl.ds(0, _EMBED), pl.ds(b0, _BBLK)],
            sem_o,
        )

    def wait_o(buf):
        pltpu.make_async_copy(
            buf,
            out_hbm.at[pl.ds(0, _LCH), pl.ds(0, _EMBED), pl.ds(b0, _BBLK)],
            sem_o,
        ).wait()

    def compute(s_v, rows_v):
        @plsc.parallel_loop(0, _LCH * _NG, 1, unroll=2)
        def _loop(i):
            li = lax.shift_right_logical(i, 5)
            g = lax.bitwise_and(i, _NG - 1)
            s = s_v[li, pl.ds(g * _LANES, _LANES)]
            base = ((s - _LOWER) / _STEP).astype(jnp.int32) * _EMBED
            vals = [
                plsc.load_gather(table_v, [base + col])
                for col in range(_EMBED)
            ]
            for col in range(_EMBED):
                rows_v[li, col, pl.ds(g * _LANES, _LANES)] = vals[col]

    start_s(0, s_bufs[0])
    start_s(1, s_bufs[1])
    for k in range(2):
        wait_s(s_bufs[k])
        compute(s_bufs[k], rows_bufs[k])
        start_o(k, rows_bufs[k])
        start_s(k + 2, s_bufs[k])

    def pair_body(pi, carry):
        for k in range(2):
            ci = pi * 2 + k
            wait_s(s_bufs[k])
            wait_o(rows_bufs[k])
            compute(s_bufs[k], rows_bufs[k])
            start_o(ci, rows_bufs[k])

            @pl.when(ci + 2 < _NCHUNK)
            def _():
                start_s(ci + 2, s_bufs[k])

        return carry

    lax.fori_loop(1, _NCHUNK // 2, pair_body, 0)
    wait_o(rows_bufs[0])
    wait_o(rows_bufs[1])


def kernel(scores, table):
    f = pl.kernel(
        _body,
        out_type=jax.ShapeDtypeStruct((_L, _EMBED, _B), jnp.float32),
        mesh=plsc.VectorSubcoreMesh(core_axis_name="c", subcore_axis_name="s"),
        compiler_params=pltpu.CompilerParams(needs_layout_passes=False),
        scratch_types=[
            [pltpu.VMEM((_LCH, _BBLK), jnp.float32) for _ in range(2)],
            [pltpu.VMEM((_LCH, _EMBED, _BBLK), jnp.float32) for _ in range(2)],
            pltpu.VMEM((_NUM_HISTOGRAM * _EMBED,), jnp.float32),
            pltpu.SemaphoreType.DMA,
            pltpu.SemaphoreType.DMA,
        ],
    )
    out_t = f(scores.T, table.reshape(_NUM_HISTOGRAM * _EMBED))
    return jnp.transpose(out_t, (2, 0, 1))
